# Initial kernel scaffold; baseline (speedup 1.0000x reference)
import jax
import jax.numpy as jnp
from jax.experimental import pallas as pl


def kernel(hist_bert, flat_idx, segment_ids, x801, w, distance_feat, W1, b1, W2, b2, q, E1, e1, E2, e2, M1, m1, M2, m2):
    raise NotImplementedError("write your pallas kernel here")


# speedup vs baseline: 5.5809x; 5.5809x over previous
import functools

import jax
import jax.numpy as jnp
from jax import lax
from jax.experimental import pallas as pl
from jax.experimental.pallas import tpu as pltpu
from jax.experimental.pallas import tpu_sc as plsc

N_HIST = 100000
HB = 768
H1 = 256
D = 128
A = 4
B = 4096
T = 524288
BIN = 801
NSEG = B * A
ROWW = 144

NW = 32
SEGW = NSEG // NW
TB = 256
MROW = 1000
HROW = 512


def _mlp_body(a_ref, w1_ref, b1_ref, w2c_ref, b2c_ref, out_ref):
    z = jnp.dot(a_ref[...], w1_ref[...], preferred_element_type=jnp.float32)
    z = jnp.maximum(z + b1_ref[...], 0.0)
    out_ref[...] = (
        jnp.dot(z, w2c_ref[...], preferred_element_type=jnp.float32)
        + b2c_ref[...]
    )


def _hist_mlp(hist_bert, W1, b1, W2cat, b2cat):
    grid = N_HIST // MROW
    return pl.pallas_call(
        _mlp_body,
        grid=(grid,),
        in_specs=[
            pl.BlockSpec((MROW, HB), lambda i: (i, 0)),
            pl.BlockSpec((HB, H1), lambda i: (0, 0)),
            pl.BlockSpec((1, H1), lambda i: (0, 0)),
            pl.BlockSpec((H1, ROWW), lambda i: (0, 0)),
            pl.BlockSpec((1, ROWW), lambda i: (0, 0)),
        ],
        out_specs=pl.BlockSpec((MROW, ROWW), lambda i: (i, 0)),
        out_shape=jax.ShapeDtypeStruct((N_HIST, ROWW), jnp.float32),
    )(hist_bert, W1, b1, W2cat, b2cat)


def _sc_pool_body(tab_hbm, fi_hbm, sg_hbm, ss_hbm, out_hbm,
                  fi_v, sg_v, rows_v, acc_v, den_v,
                  sa_v, sb_v, sem0, sem1):
    wid = lax.axis_index("c") * 16 + lax.axis_index("s")
    seg0 = wid * SEGW

    pltpu.sync_copy(ss_hbm.at[pl.ds(seg0, 16)], sa_v)
    pltpu.sync_copy(ss_hbm.at[pl.ds(seg0 + SEGW, 16)], sb_v)
    t0 = sa_v[...][0]
    t1 = sb_v[...][0]

    zf = jnp.zeros((16,), jnp.float32)

    def _zero(r, _):
        for c in range(8):
            acc_v[r, pl.ds(c * 16, 16)] = zf
        den_v[r, :] = zf
        return _

    lax.fori_loop(0, SEGW, _zero, None)

    lane = lax.iota(jnp.int32, 16)

    def _block(kb, _):
        tb = kb * TB
        pltpu.sync_copy(fi_hbm.at[pl.ds(kb * 2, 2)], fi_v)
        cp0 = pltpu.async_copy(tab_hbm.at[fi_v.at[0]],
                               rows_v.at[pl.ds(0, 128)], sem0)
        cp1 = pltpu.async_copy(tab_hbm.at[fi_v.at[1]],
                               rows_v.at[pl.ds(128, 128)], sem1)
        pltpu.sync_copy(sg_hbm.at[pl.ds(tb, TB)], sg_v)
        cp0.wait()
        cp1.wait()

        jlo = jnp.maximum(t0 - tb, 0)
        jhi = jnp.minimum(t1 - tb, TB)

        def _tok(j, _):
            idxj = jnp.full((16,), j, jnp.int32)
            sgb = plsc.load_gather(sg_v, [idxj])
            r = sgb[0] - seg0
            asp = jnp.bitwise_and(sgb, A - 1)
            lg = rows_v[j, pl.ds(D, 16)]
            s = jnp.sum(jnp.where(lane == asp, lg, 0.0))
            eb = jnp.exp(jnp.full((16,), s, jnp.float32))
            for c in range(8):
                sl = pl.ds(c * 16, 16)
                plsc.addupdate(acc_v.at[r, sl], rows_v[j, sl] * eb)
            plsc.addupdate(den_v.at[r, :], eb)
            return _

        lax.fori_loop(jlo, jhi, _tok, None)
        return _

    kb0 = t0 // TB
    kb1 = (t1 + (TB - 1)) // TB
    lax.fori_loop(kb0, kb1, _block, None)

    def _scale(r, _):
        dv = den_v[r, :]
        rb = jnp.where(dv > 0.0, 1.0 / dv, 0.0)
        for c in range(8):
            sl = pl.ds(c * 16, 16)
            acc_v[r, sl] = acc_v[r, sl] * rb
        return _

    lax.fori_loop(0, SEGW, _scale, None)
    pltpu.sync_copy(acc_v, out_hbm.at[pl.ds(seg0, SEGW)])


def _sc_pool(row_tab, flat_idx2, segment_ids, seg_starts):
    mesh = plsc.VectorSubcoreMesh(core_axis_name="c", subcore_axis_name="s")
    f = pl.kernel(
        _sc_pool_body,
        out_type=jax.ShapeDtypeStruct((NSEG, D), jnp.float32),
        mesh=mesh,
        scratch_types=[
            pltpu.VMEM((2, 128), jnp.int32),
            pltpu.VMEM((TB,), jnp.int32),
            pltpu.VMEM((TB, ROWW), jnp.float32),
            pltpu.VMEM((SEGW, D), jnp.float32),
            pltpu.VMEM((SEGW, 16), jnp.float32),
            pltpu.VMEM((16,), jnp.int32),
            pltpu.VMEM((16,), jnp.int32),
            pltpu.SemaphoreType.DMA,
            pltpu.SemaphoreType.DMA,
        ],
        compiler_params=pltpu.CompilerParams(use_tc_tiling_on_sc=False,
                                             needs_layout_passes=False),
    )
    return f(row_tab, flat_idx2, segment_ids, seg_starts)


def _head_body(t2_ref, x_ref, w_ref, d_ref, e1w_ref, e1b_ref, e2w_ref,
               e2b_ref, m1a_ref, m1b_ref, m1bias_ref, m2_ref, m2b_ref,
               out_ref):
    z = jnp.dot(x_ref[...], e1w_ref[...], preferred_element_type=jnp.float32)
    z = jnp.maximum(z + e1b_ref[...], 0.0)
    bv = jnp.dot(z, e2w_ref[...], preferred_element_type=jnp.float32)
    bv = bv + e2b_ref[...]
    t2 = t2_ref[...]
    wb = w_ref[...]
    wt = (t2[:, 0 * D:1 * D] * wb[:, 0:1] + t2[:, 1 * D:2 * D] * wb[:, 1:2]
          + t2[:, 2 * D:3 * D] * wb[:, 2:3] + t2[:, 3 * D:4 * D] * wb[:, 3:4])
    m = wt * bv
    h1 = jnp.dot(m, m1a_ref[...], preferred_element_type=jnp.float32)
    h1 = jnp.maximum(h1 + d_ref[...] * m1b_ref[...] + m1bias_ref[...], 0.0)
    out_ref[...] = (
        jnp.dot(h1, m2_ref[...], preferred_element_type=jnp.float32)
        + m2b_ref[...]
    )


def _head(taste2, x801, w, dist, E1, e1, E2, e2, M1a, M1b, m1, M2, m2):
    grid = B // HROW
    return pl.pallas_call(
        _head_body,
        grid=(grid,),
        in_specs=[
            pl.BlockSpec((HROW, A * D), lambda i: (i, 0)),
            pl.BlockSpec((HROW, BIN), lambda i: (i, 0)),
            pl.BlockSpec((HROW, A), lambda i: (i, 0)),
            pl.BlockSpec((HROW, 1), lambda i: (i, 0)),
            pl.BlockSpec((BIN, H1), lambda i: (0, 0)),
            pl.BlockSpec((1, H1), lambda i: (0, 0)),
            pl.BlockSpec((H1, D), lambda i: (0, 0)),
            pl.BlockSpec((1, D), lambda i: (0, 0)),
            pl.BlockSpec((D, 64), lambda i: (0, 0)),
            pl.BlockSpec((1, 64), lambda i: (0, 0)),
            pl.BlockSpec((1, 64), lambda i: (0, 0)),
            pl.BlockSpec((64, 1), lambda i: (0, 0)),
            pl.BlockSpec((1, 1), lambda i: (0, 0)),
        ],
        out_specs=pl.BlockSpec((HROW, 1), lambda i: (i, 0)),
        out_shape=jax.ShapeDtypeStruct((B, 1), jnp.float32),
    )(taste2, x801, w, dist, E1, e1, E2, e2, M1a, M1b, m1, M2, m2)


def kernel(hist_bert, flat_idx, segment_ids, x801, w, distance_feat,
           W1, b1, W2, b2, q, E1, e1, E2, e2, M1, m1, M2, m2):
    qt = q.T
    W2cat = jnp.pad(jnp.concatenate([W2, W2 @ qt], axis=1),
                    ((0, 0), (0, ROWW - D - A)))
    b2cat = jnp.pad(jnp.concatenate([b2, b2 @ qt]), (0, ROWW - D - A))

    row_tab = _hist_mlp(hist_bert, W1, b1.reshape(1, H1), W2cat,
                        b2cat.reshape(1, ROWW))

    ss = jnp.searchsorted(segment_ids,
                          jnp.arange(NSEG + 1, dtype=jnp.int32)).astype(jnp.int32)
    ss = jnp.concatenate([ss, jnp.full((15,), T, jnp.int32)])

    taste = _sc_pool(row_tab, flat_idx.reshape(T // 128, 128),
                     segment_ids, ss)

    out = _head(taste.reshape(B, A * D), x801, w, distance_feat.reshape(B, 1),
                E1, e1.reshape(1, H1), E2, e2.reshape(1, D),
                M1[:D], M1[D:D + 1], m1.reshape(1, 64), M2, m2.reshape(1, 1))
    return out.reshape(B)


# speedup vs baseline: 11.8584x; 2.1248x over previous
import functools

import jax
import jax.numpy as jnp
from jax import lax
from jax.experimental import pallas as pl
from jax.experimental.pallas import tpu as pltpu
from jax.experimental.pallas import tpu_sc as plsc

N_HIST = 100000
HB = 768
H1 = 256
D = 128
A = 4
B = 4096
T = 524288
BIN = 801
NSEG = B * A
ROWW = 144

NW = 32
SEGW = NSEG // NW
TB = 128
NPOS = T // 8
MROW = 1000
HROW = 512


def _mlp_body(a_ref, w1_ref, b1_ref, w2c_ref, b2c_ref, out_ref):
    z = jnp.dot(a_ref[...], w1_ref[...], preferred_element_type=jnp.float32)
    z = jnp.maximum(z + b1_ref[...], 0.0)
    out_ref[...] = (
        jnp.dot(z, w2c_ref[...], preferred_element_type=jnp.float32)
        + b2c_ref[...]
    )


def _hist_mlp(hist_bert, W1, b1, W2cat, b2cat):
    grid = N_HIST // MROW
    return pl.pallas_call(
        _mlp_body,
        grid=(grid,),
        in_specs=[
            pl.BlockSpec((MROW, HB), lambda i: (i, 0)),
            pl.BlockSpec((HB, H1), lambda i: (0, 0)),
            pl.BlockSpec((1, H1), lambda i: (0, 0)),
            pl.BlockSpec((H1, ROWW), lambda i: (0, 0)),
            pl.BlockSpec((1, ROWW), lambda i: (0, 0)),
        ],
        out_specs=pl.BlockSpec((MROW, ROWW), lambda i: (i, 0)),
        out_shape=jax.ShapeDtypeStruct((N_HIST, ROWW), jnp.float32),
    )(hist_bert, W1, b1, W2cat, b2cat)


def _sc_pool_body(tab_hbm, tabf_hbm, fi_hbm, sg_hbm, out_hbm,
                  fi0_v, ci0_v, sg0_v, sc0_v, rows0_v,
                  fi1_v, ci1_v, sg1_v, sc1_v, rows1_v,
                  acc_v, den_v, bs_v, sem0, sem1):
    wid = lax.axis_index("c") * 16 + lax.axis_index("s")
    seg0 = wid * SEGW

    def _lb8(target):
        def _step(_, lh):
            lo, hi = lh
            mid = jnp.minimum(lax.div(lo + hi, 2), NPOS - 1)
            pltpu.sync_copy(sg_hbm.at[pl.ds(mid * 8, 8)],
                            bs_v.at[pl.ds(0, 8)])
            v = bs_v[...][0]
            pred = v < target
            nlo = jnp.where(pred, mid + 1, lo)
            nhi = jnp.where(pred, hi, mid)
            done = lo >= hi
            return (jnp.where(done, lo, nlo), jnp.where(done, hi, nhi))

        lo, _ = lax.fori_loop(0, 17, _step, (jnp.int32(0), jnp.int32(NPOS)))
        return lo

    t0 = 8 * jnp.maximum(_lb8(seg0) - 1, 0)
    t1 = 8 * _lb8(seg0 + SEGW)

    zf = jnp.zeros((16,), jnp.float32)

    def _zero(r, _):
        for c in range(8):
            acc_v[r, pl.ds(c * 16, 16)] = zf
        den_v[r, :] = zf
        return _

    lax.fori_loop(0, SEGW + 1, _zero, None)

    kb0 = lax.div(t0, TB)
    nb = lax.div(t1 - kb0 * TB + (TB - 1), TB)

    def _prefetch(i, fi_b, ci_b, sg_b, sc_b, rows_b, sem):
        kb = jnp.minimum(kb0 + i, (T // TB) - 1)
        tb = kb * TB
        pltpu.sync_copy(fi_hbm.at[pl.ds(tb, TB)], fi_b)
        pltpu.sync_copy(sg_hbm.at[pl.ds(tb, TB)], sg_b)
        for c in range(TB // 16):
            sl = pl.ds(c * 16, 16)
            ci_b[sl] = fi_b[sl] * ROWW + (jnp.bitwise_and(sg_b[sl], A - 1) + D)
        pltpu.async_copy(tab_hbm.at[fi_b], rows_b, sem)
        pltpu.async_copy(tabf_hbm.at[ci_b], sc_b, sem)

    def _process(fi_b, ci_b, sg_b, sc_b, rows_b, sem):
        pltpu.make_async_copy(tab_hbm.at[fi_b], rows_b, sem).wait()
        pltpu.make_async_copy(tabf_hbm.at[ci_b], sc_b, sem).wait()
        for c in range(TB // 16):
            sl = pl.ds(c * 16, 16)
            sc_b[sl] = plsc.bitcast(
                jnp.exp(plsc.bitcast(sc_b[sl], jnp.float32)), jnp.int32)

        def _tok(j, _):
            idxj = jnp.full((16,), j, jnp.int32)
            sgb = plsc.load_gather(sg_b, [idxj])
            eb = plsc.bitcast(plsc.load_gather(sc_b, [idxj]), jnp.float32)
            r = sgb[0] - seg0
            rc = jnp.where((r >= 0) & (r < SEGW), r, SEGW)
            for c in range(8):
                sl = pl.ds(c * 16, 16)
                plsc.addupdate(acc_v.at[rc, sl], rows_b[j, sl] * eb)
            plsc.addupdate(den_v.at[rc, :], eb)
            return _

        lax.fori_loop(0, TB, _tok, None)

    bufs = ((fi0_v, ci0_v, sg0_v, sc0_v, rows0_v, sem0),
            (fi1_v, ci1_v, sg1_v, sc1_v, rows1_v, sem1))

    @pl.when(nb > 0)
    def _():
        _prefetch(0, *bufs[0])

    def _pair(i, _):
        even = jnp.bitwise_and(i, 1) == 0

        @pl.when(even)
        def _():
            @pl.when(i + 1 < nb)
            def _():
                _prefetch(i + 1, *bufs[1])
            _process(*bufs[0])

        @pl.when(jnp.logical_not(even))
        def _():
            @pl.when(i + 1 < nb)
            def _():
                _prefetch(i + 1, *bufs[0])
            _process(*bufs[1])

        return _

    lax.fori_loop(0, nb, _pair, None)

    def _scale(r, _):
        dv = den_v[r, :]
        rb = jnp.where(dv > 0.0, 1.0 / dv, 0.0)
        for c in range(8):
            sl = pl.ds(c * 16, 16)
            acc_v[r, sl] = acc_v[r, sl] * rb
        return _

    lax.fori_loop(0, SEGW, _scale, None)
    pltpu.sync_copy(acc_v.at[pl.ds(0, SEGW)], out_hbm.at[pl.ds(seg0, SEGW)])


def _sc_pool(row_tab, row_tab_flat, flat_idx, segment_ids):
    mesh = plsc.VectorSubcoreMesh(core_axis_name="c", subcore_axis_name="s")
    buf = [
        pltpu.VMEM((TB,), jnp.int32),
        pltpu.VMEM((TB,), jnp.int32),
        pltpu.VMEM((TB,), jnp.int32),
        pltpu.VMEM((TB,), jnp.int32),
        pltpu.VMEM((TB, ROWW), jnp.float32),
    ]
    f = pl.kernel(
        _sc_pool_body,
        out_type=jax.ShapeDtypeStruct((NSEG, D), jnp.float32),
        mesh=mesh,
        scratch_types=buf + buf + [
            pltpu.VMEM((SEGW + 1, D), jnp.float32),
            pltpu.VMEM((SEGW + 1, 16), jnp.float32),
            pltpu.VMEM((16,), jnp.int32),
            pltpu.SemaphoreType.DMA,
            pltpu.SemaphoreType.DMA,
        ],
        compiler_params=pltpu.CompilerParams(use_tc_tiling_on_sc=False,
                                             needs_layout_passes=False),
    )
    return f(row_tab, row_tab_flat, flat_idx, segment_ids)


def _head_body(t2_ref, x_ref, w_ref, d_ref, e1w_ref, e1b_ref, e2w_ref,
               e2b_ref, m1a_ref, m1b_ref, m1bias_ref, m2_ref, m2b_ref,
               out_ref):
    z = jnp.dot(x_ref[...], e1w_ref[...], preferred_element_type=jnp.float32)
    z = jnp.maximum(z + e1b_ref[...], 0.0)
    bv = jnp.dot(z, e2w_ref[...], preferred_element_type=jnp.float32)
    bv = bv + e2b_ref[...]
    t2 = t2_ref[...]
    wb = w_ref[...]
    wt = (t2[:, 0 * D:1 * D] * wb[:, 0:1] + t2[:, 1 * D:2 * D] * wb[:, 1:2]
          + t2[:, 2 * D:3 * D] * wb[:, 2:3] + t2[:, 3 * D:4 * D] * wb[:, 3:4])
    m = wt * bv
    h1 = jnp.dot(m, m1a_ref[...], preferred_element_type=jnp.float32)
    h1 = jnp.maximum(h1 + d_ref[...] * m1b_ref[...] + m1bias_ref[...], 0.0)
    out_ref[...] = (
        jnp.dot(h1, m2_ref[...], preferred_element_type=jnp.float32)
        + m2b_ref[...]
    )


def _head(taste2, x801, w, dist, E1, e1, E2, e2, M1a, M1b, m1, M2, m2):
    grid = B // HROW
    return pl.pallas_call(
        _head_body,
        grid=(grid,),
        in_specs=[
            pl.BlockSpec((HROW, A * D), lambda i: (i, 0)),
            pl.BlockSpec((HROW, BIN), lambda i: (i, 0)),
            pl.BlockSpec((HROW, A), lambda i: (i, 0)),
            pl.BlockSpec((HROW, 1), lambda i: (i, 0)),
            pl.BlockSpec((BIN, H1), lambda i: (0, 0)),
            pl.BlockSpec((1, H1), lambda i: (0, 0)),
            pl.BlockSpec((H1, D), lambda i: (0, 0)),
            pl.BlockSpec((1, D), lambda i: (0, 0)),
            pl.BlockSpec((D, 64), lambda i: (0, 0)),
            pl.BlockSpec((1, 64), lambda i: (0, 0)),
            pl.BlockSpec((1, 64), lambda i: (0, 0)),
            pl.BlockSpec((64, 1), lambda i: (0, 0)),
            pl.BlockSpec((1, 1), lambda i: (0, 0)),
        ],
        out_specs=pl.BlockSpec((HROW, 1), lambda i: (i, 0)),
        out_shape=jax.ShapeDtypeStruct((B, 1), jnp.float32),
    )(taste2, x801, w, dist, E1, e1, E2, e2, M1a, M1b, m1, M2, m2)


def kernel(hist_bert, flat_idx, segment_ids, x801, w, distance_feat,
           W1, b1, W2, b2, q, E1, e1, E2, e2, M1, m1, M2, m2):
    qt = q.T
    W2cat = jnp.pad(jnp.concatenate([W2, W2 @ qt], axis=1),
                    ((0, 0), (0, ROWW - D - A)))
    b2cat = jnp.pad(jnp.concatenate([b2, b2 @ qt]), (0, ROWW - D - A))

    row_tab = _hist_mlp(hist_bert, W1, b1.reshape(1, H1), W2cat,
                        b2cat.reshape(1, ROWW))

    row_tab_flat = jax.lax.bitcast_convert_type(
        row_tab, jnp.int32).reshape(N_HIST * ROWW)
    taste = _sc_pool(row_tab, row_tab_flat, flat_idx, segment_ids)

    out = _head(taste.reshape(B, A * D), x801, w, distance_feat.reshape(B, 1),
                E1, e1.reshape(1, H1), E2, e2.reshape(1, D),
                M1[:D], M1[D:D + 1], m1.reshape(1, 64), M2, m2.reshape(1, 1))
    return out.reshape(B)


# speedup vs baseline: 31.0392x; 2.6175x over previous
import functools

import jax
import jax.numpy as jnp
from jax import lax
from jax.experimental import pallas as pl
from jax.experimental.pallas import tpu as pltpu
from jax.experimental.pallas import tpu_sc as plsc

N_HIST = 100000
HB = 768
H1 = 256
D = 128
A = 4
B = 4096
T = 524288
BIN = 801
NSEG = B * A

NW = 32
SEGW = NSEG // NW
TB = 128
NPOS = T // 8
MROW = 1024
HROW = 512


def _mlp_body(a_ref, w1_ref, b1_ref, w2_ref, b2_ref, q_ref, h_ref, lg_ref):
    z = jnp.dot(a_ref[...], w1_ref[...], preferred_element_type=jnp.float32)
    z = jnp.maximum(z + b1_ref[...], 0.0)
    h = jnp.dot(z, w2_ref[...], preferred_element_type=jnp.float32)
    h = h + b2_ref[...]
    h_ref[...] = h
    lg_ref[...] = lax.dot_general(q_ref[...], h, (((1,), (1,)), ((), ())),
                                  preferred_element_type=jnp.float32)


def _hist_mlp(hist_bert, W1, b1, W2, b2, q):
    grid = (N_HIST + MROW - 1) // MROW
    return pl.pallas_call(
        _mlp_body,
        grid=(grid,),
        in_specs=[
            pl.BlockSpec((MROW, HB), lambda i: (i, 0)),
            pl.BlockSpec((HB, H1), lambda i: (0, 0)),
            pl.BlockSpec((1, H1), lambda i: (0, 0)),
            pl.BlockSpec((H1, D), lambda i: (0, 0)),
            pl.BlockSpec((1, D), lambda i: (0, 0)),
            pl.BlockSpec((A, D), lambda i: (0, 0)),
        ],
        out_specs=[
            pl.BlockSpec((MROW, D), lambda i: (i, 0)),
            pl.BlockSpec((A, MROW), lambda i: (0, i)),
        ],
        out_shape=[
            jax.ShapeDtypeStruct((N_HIST, D), jnp.float32),
            jax.ShapeDtypeStruct((A, N_HIST), jnp.float32),
        ],
    )(hist_bert, W1, b1, W2, b2, q)


def _sc_pool_body(tab_hbm, tabf_hbm, fi_hbm, sg_hbm, out_hbm,
                  fi0_v, ci0_v, sg0_v, sc0_v, rows0_v,
                  fi1_v, ci1_v, sg1_v, sc1_v, rows1_v,
                  acc_v, den_v, bs_v, sem0, sem1):
    wid = lax.axis_index("c") * 16 + lax.axis_index("s")
    seg0 = wid * SEGW

    def _lb8(target):
        def _step(_, lh):
            lo, hi = lh
            mid = jnp.minimum(lax.div(lo + hi, 2), NPOS - 1)
            pltpu.sync_copy(sg_hbm.at[pl.ds(mid * 8, 8)],
                            bs_v.at[pl.ds(0, 8)])
            v = bs_v[...][0]
            pred = v < target
            nlo = jnp.where(pred, mid + 1, lo)
            nhi = jnp.where(pred, hi, mid)
            done = lo >= hi
            return (jnp.where(done, lo, nlo), jnp.where(done, hi, nhi))

        lo, _ = lax.fori_loop(0, 17, _step, (jnp.int32(0), jnp.int32(NPOS)))
        return lo

    t0 = 8 * jnp.maximum(_lb8(seg0) - 1, 0)
    t1 = 8 * _lb8(seg0 + SEGW)

    zf = jnp.zeros((16,), jnp.float32)

    def _zero(r, _):
        for c in range(8):
            acc_v[r, pl.ds(c * 16, 16)] = zf
        den_v[r, :] = zf
        return _

    lax.fori_loop(0, SEGW + 1, _zero, None)

    kb0 = lax.div(t0, TB)
    nb = lax.div(t1 - kb0 * TB + (TB - 1), TB)

    def _prefetch(i, fi_b, ci_b, sg_b, sc_b, rows_b, sem):
        kb = jnp.minimum(kb0 + i, (T // TB) - 1)
        tb = kb * TB
        pltpu.sync_copy(fi_hbm.at[pl.ds(tb, TB)], fi_b)
        pltpu.sync_copy(sg_hbm.at[pl.ds(tb, TB)], sg_b)
        for c in range(TB // 16):
            sl = pl.ds(c * 16, 16)
            ci_b[sl] = (jnp.bitwise_and(sg_b[sl], A - 1) * N_HIST
                        + fi_b[sl])
        pltpu.async_copy(tab_hbm.at[fi_b], rows_b, sem)
        pltpu.async_copy(tabf_hbm.at[ci_b], sc_b, sem)

    def _process(fi_b, ci_b, sg_b, sc_b, rows_b, sem):
        pltpu.make_async_copy(tab_hbm.at[fi_b], rows_b, sem).wait()
        pltpu.make_async_copy(tabf_hbm.at[ci_b], sc_b, sem).wait()
        for c in range(TB // 16):
            sl = pl.ds(c * 16, 16)
            sc_b[sl] = jnp.exp(sc_b[sl])

        @plsc.parallel_loop(0, TB, unroll=4)
        def _tok(j):
            idxj = jnp.full((16,), j, jnp.int32)
            sgb = plsc.load_gather(sg_b, [idxj])
            eb = plsc.load_gather(sc_b, [idxj])
            r = sgb[0] - seg0
            rc = jnp.where((r >= 0) & (r < SEGW), r, SEGW)
            for c in range(8):
                sl = pl.ds(c * 16, 16)
                plsc.addupdate(acc_v.at[rc, sl], rows_b[j, sl] * eb)
            plsc.addupdate(den_v.at[rc, :], eb)

    bufs = ((fi0_v, ci0_v, sg0_v, sc0_v, rows0_v, sem0),
            (fi1_v, ci1_v, sg1_v, sc1_v, rows1_v, sem1))

    @pl.when(nb > 0)
    def _():
        _prefetch(0, *bufs[0])

    def _pair(i, _):
        even = jnp.bitwise_and(i, 1) == 0

        @pl.when(even)
        def _():
            @pl.when(i + 1 < nb)
            def _():
                _prefetch(i + 1, *bufs[1])
            _process(*bufs[0])

        @pl.when(jnp.logical_not(even))
        def _():
            @pl.when(i + 1 < nb)
            def _():
                _prefetch(i + 1, *bufs[0])
            _process(*bufs[1])

        return _

    lax.fori_loop(0, nb, _pair, None)

    def _scale(r, _):
        dv = den_v[r, :]
        rb = jnp.where(dv > 0.0, 1.0 / dv, 0.0)
        for c in range(8):
            sl = pl.ds(c * 16, 16)
            acc_v[r, sl] = acc_v[r, sl] * rb
        return _

    lax.fori_loop(0, SEGW, _scale, None)
    pltpu.sync_copy(acc_v.at[pl.ds(0, SEGW)], out_hbm.at[pl.ds(seg0, SEGW)])


def _sc_pool(row_tab, row_tab_flat, flat_idx, segment_ids):
    mesh = plsc.VectorSubcoreMesh(core_axis_name="c", subcore_axis_name="s")
    buf = [
        pltpu.VMEM((TB,), jnp.int32),
        pltpu.VMEM((TB,), jnp.int32),
        pltpu.VMEM((TB,), jnp.int32),
        pltpu.VMEM((TB,), jnp.float32),
        pltpu.VMEM((TB, D), jnp.float32),
    ]
    f = pl.kernel(
        _sc_pool_body,
        out_type=jax.ShapeDtypeStruct((NSEG, D), jnp.float32),
        mesh=mesh,
        scratch_types=buf + buf + [
            pltpu.VMEM((SEGW + 1, D), jnp.float32),
            pltpu.VMEM((SEGW + 1, 16), jnp.float32),
            pltpu.VMEM((16,), jnp.int32),
            pltpu.SemaphoreType.DMA,
            pltpu.SemaphoreType.DMA,
        ],
        compiler_params=pltpu.CompilerParams(use_tc_tiling_on_sc=False,
                                             needs_layout_passes=False),
    )
    return f(row_tab, row_tab_flat, flat_idx, segment_ids)


def _head_body(t2_ref, x_ref, w_ref, d_ref, e1w_ref, e1b_ref, e2w_ref,
               e2b_ref, m1a_ref, m1b_ref, m1bias_ref, m2_ref, m2b_ref,
               out_ref):
    z = jnp.dot(x_ref[...], e1w_ref[...], preferred_element_type=jnp.float32)
    z = jnp.maximum(z + e1b_ref[...], 0.0)
    bv = jnp.dot(z, e2w_ref[...], preferred_element_type=jnp.float32)
    bv = bv + e2b_ref[...]
    t2 = t2_ref[...]
    wb = w_ref[...]
    wt = (t2[:, 0 * D:1 * D] * wb[:, 0:1] + t2[:, 1 * D:2 * D] * wb[:, 1:2]
          + t2[:, 2 * D:3 * D] * wb[:, 2:3] + t2[:, 3 * D:4 * D] * wb[:, 3:4])
    m = wt * bv
    h1 = jnp.dot(m, m1a_ref[...], preferred_element_type=jnp.float32)
    h1 = jnp.maximum(h1 + d_ref[...] * m1b_ref[...] + m1bias_ref[...], 0.0)
    out_ref[...] = (
        jnp.dot(h1, m2_ref[...], preferred_element_type=jnp.float32)
        + m2b_ref[...]
    )


def _head(taste2, x801, w, dist, E1, e1, E2, e2, M1a, M1b, m1, M2, m2):
    grid = B // HROW
    return pl.pallas_call(
        _head_body,
        grid=(grid,),
        in_specs=[
            pl.BlockSpec((HROW, A * D), lambda i: (i, 0)),
            pl.BlockSpec((HROW, BIN), lambda i: (i, 0)),
            pl.BlockSpec((HROW, A), lambda i: (i, 0)),
            pl.BlockSpec((HROW, 1), lambda i: (i, 0)),
            pl.BlockSpec((BIN, H1), lambda i: (0, 0)),
            pl.BlockSpec((1, H1), lambda i: (0, 0)),
            pl.BlockSpec((H1, D), lambda i: (0, 0)),
            pl.BlockSpec((1, D), lambda i: (0, 0)),
            pl.BlockSpec((D, 64), lambda i: (0, 0)),
            pl.BlockSpec((1, 64), lambda i: (0, 0)),
            pl.BlockSpec((1, 64), lambda i: (0, 0)),
            pl.BlockSpec((64, 1), lambda i: (0, 0)),
            pl.BlockSpec((1, 1), lambda i: (0, 0)),
        ],
        out_specs=pl.BlockSpec((HROW, 1), lambda i: (i, 0)),
        out_shape=jax.ShapeDtypeStruct((B, 1), jnp.float32),
    )(taste2, x801, w, dist, E1, e1, E2, e2, M1a, M1b, m1, M2, m2)


def kernel(hist_bert, flat_idx, segment_ids, x801, w, distance_feat,
           W1, b1, W2, b2, q, E1, e1, E2, e2, M1, m1, M2, m2):
    h_tab, lgT = _hist_mlp(hist_bert, W1, b1.reshape(1, H1), W2,
                           b2.reshape(1, D), q)
    taste = _sc_pool(h_tab, lgT.reshape(A * N_HIST), flat_idx, segment_ids)

    out = _head(taste.reshape(B, A * D), x801, w, distance_feat.reshape(B, 1),
                E1, e1.reshape(1, H1), E2, e2.reshape(1, D),
                M1[:D], M1[D:D + 1], m1.reshape(1, 64), M2, m2.reshape(1, 1))
    return out.reshape(B)
